# baseline (speedup 1.0000x reference)
import functools

import jax
import jax.numpy as jnp
from jax import lax
from jax.experimental import pallas as pl
from jax.experimental.pallas import tpu as pltpu
from jax.experimental.pallas import tpu_sc as plsc

_TILE = 64
_FFB = 256
_NW = 32
_CHUNK = 32


def _router_body(x_ref, gw_ref, wq_ref, pos_ref, te_ref, aux_ref):
    x = x_ref[...]
    gw = gw_ref[...]
    n_tok = x.shape[0]
    n_exp = gw.shape[0]
    logits = jax.lax.dot_general(x, gw, (((1,), (1,)), ((), ())),
                                 preferred_element_type=jnp.float32)
    mx = jnp.max(logits, axis=-1, keepdims=True)
    ex = jnp.exp(logits - mx)
    probs = ex / jnp.sum(ex, axis=-1, keepdims=True)
    iot = jax.lax.broadcasted_iota(jnp.int32, probs.shape, 1)
    m1 = jnp.max(probs, axis=-1, keepdims=True)
    a1 = jnp.min(jnp.where(probs == m1, iot, n_exp), axis=-1, keepdims=True)
    sel1 = iot == a1
    pm = jnp.where(sel1, -1.0, probs)
    m2 = jnp.max(pm, axis=-1, keepdims=True)
    a2 = jnp.min(jnp.where(pm == m2, iot, n_exp), axis=-1, keepdims=True)
    sel2 = iot == a2
    wsum = m1 + m2
    cnt = jnp.sum(sel1.astype(jnp.float32) + sel2.astype(jnp.float32),
                  axis=0, keepdims=True)
    pmean = jnp.mean(probs, axis=0, keepdims=True)
    f = cnt / (n_tok * 2.0)
    aux_ref[...] = jnp.sum(f * pmean, keepdims=True).reshape(1, 1) * n_exp

    lanes = 128
    rws = n_tok // lanes
    ea = jnp.concatenate([jnp.reshape(a1, (rws, lanes)),
                          jnp.reshape(a2, (rws, lanes))], axis=0)
    wq_ref[...] = jnp.concatenate(
        [jnp.reshape(m1 / wsum, (rws, lanes)),
         jnp.reshape(m2 / wsum, (rws, lanes))], axis=0)
    rows = 2 * rws
    n_tiles = te_ref.shape[1]

    ci = cnt.astype(jnp.int32)
    pc = ((ci + _TILE - 1) // _TILE) * _TILE
    run = jnp.zeros((1, 1), jnp.int32)
    poffs, pcums = [], []
    for e in range(n_exp):
        poffs.append(run)
        run = run + pc[:, e:e + 1]
        pcums.append(run)

    li = lax.broadcasted_iota(jnp.int32, (lanes, lanes), 0)
    lj = lax.broadcasted_iota(jnp.int32, (lanes, lanes), 1)
    lt_inc = (li <= lj).astype(jnp.float32)
    ri = lax.broadcasted_iota(jnp.int32, (rows, rows), 0)
    rj = lax.broadcasted_iota(jnp.int32, (rows, rows), 1)
    rt_exc = (ri > rj).astype(jnp.float32)

    pos = jnp.zeros(ea.shape, jnp.int32)
    for e in range(n_exp):
        m = (ea == e).astype(jnp.float32)
        lane_c = jnp.dot(m, lt_inc,
                         preferred_element_type=jnp.float32)
        row_tot = lane_c[:, lanes - 1:lanes]
        row_off = jnp.dot(rt_exc, row_tot,
                          preferred_element_type=jnp.float32)
        rank = (row_off + lane_c - 1.0).astype(jnp.int32)
        pos = jnp.where(ea == e, poffs[e] + rank, pos)
    pos_ref[...] = pos

    poff_row = jnp.concatenate(poffs, axis=1) // _TILE
    ptl_row = pc // _TILE
    te_ref[...] = jnp.concatenate([poff_row, ptl_row], axis=0)


def _gmm_body(tr_ref, xs_ref, w1_ref, b1_ref, w2_ref, b2_ref, ws_ref, os_ref):
    e = pl.program_id(0)
    ft = pl.program_id(1)
    w1c = w1_ref[0]
    b1c = b1_ref[0, 0]
    w2c = w2_ref[0]
    b2c = b2_ref[0]
    start = tr_ref[0, e]
    ntl = tr_ref[1, e]

    def mbody(i, _):
        base = pl.multiple_of((start + i) * _TILE, _TILE)
        sl = pl.ds(base, _TILE)
        xt = xs_ref[sl, :].astype(jnp.float32)
        h = jax.lax.dot_general(xt, w1c, (((1,), (1,)), ((), ())),
                                preferred_element_type=jnp.float32)
        h = jnp.maximum(h + b1c, 0.0)
        contrib = jax.lax.dot_general(h, w2c, (((1,), (1,)), ((), ())),
                                      preferred_element_type=jnp.float32)
        ws = ws_ref[sl, :]

        @pl.when(ft == 0)
        def _init():
            os_ref[sl, :] = (contrib + b2c) * ws

        @pl.when(ft != 0)
        def _acc():
            os_ref[sl, :] += contrib * ws

        return 0

    lax.fori_loop(0, ntl, mbody, 0)


def _make_scatter(n_tok, dm, mp, rws):
    mesh = plsc.VectorSubcoreMesh(core_axis_name="c", subcore_axis_name="s")
    tok_pw = n_tok // _NW
    ck = 64

    @functools.partial(
        pl.kernel, mesh=mesh,
        out_type=(jax.ShapeDtypeStruct((mp, dm), jnp.float32),
                  jax.ShapeDtypeStruct((mp,), jnp.float32)),
        scratch_types=[
            pltpu.VMEM((ck, dm), jnp.float32),
            pltpu.VMEM((ck,), jnp.int32),
            pltpu.VMEM((ck,), jnp.int32),
            pltpu.VMEM((ck,), jnp.float32),
            pltpu.VMEM((ck,), jnp.float32),
            pltpu.SemaphoreType.DMA,
            pltpu.SemaphoreType.DMA,
        ],
    )
    def scatter(x_hbm, pos_hbm, wq_hbm, xs_hbm, ws_hbm,
                xbuf, idx0, idx1, wb0, wb1, lsem, ssem):
        wid = lax.axis_index("s") * 2 + lax.axis_index("c")
        for c in range(tok_pw // ck):
            tb = wid * tok_pw + c * ck
            lk = c * ck
            pltpu.sync_copy(x_hbm.at[pl.ds(tb, ck)], xbuf)
            pltpu.sync_copy(pos_hbm.at[wid, pl.ds(lk, ck)], idx0)
            pltpu.sync_copy(pos_hbm.at[rws + wid, pl.ds(lk, ck)], idx1)
            pltpu.sync_copy(wq_hbm.at[wid, pl.ds(lk, ck)], wb0)
            pltpu.sync_copy(wq_hbm.at[rws + wid, pl.ds(lk, ck)], wb1)
            s1 = pltpu.async_copy(xbuf, xs_hbm.at[idx0], ssem)
            s2 = pltpu.async_copy(xbuf, xs_hbm.at[idx1], ssem)
            s3 = pltpu.async_copy(wb0, ws_hbm.at[idx0], ssem)
            s4 = pltpu.async_copy(wb1, ws_hbm.at[idx1], ssem)
            for h in (s1, s2, s3, s4):
                h.wait()

    return scatter


def _make_combine(n_tok, dm, mp):
    mesh = plsc.VectorSubcoreMesh(core_axis_name="c", subcore_axis_name="s")
    tok_pw = n_tok // _NW

    @functools.partial(
        pl.kernel, mesh=mesh,
        out_type=jax.ShapeDtypeStruct((n_tok, dm), jnp.float32),
        scratch_types=[
            pltpu.VMEM((_CHUNK, dm), jnp.float32),
            pltpu.VMEM((_CHUNK, dm), jnp.float32),
            pltpu.VMEM((_CHUNK, dm), jnp.float32),
            pltpu.VMEM((_CHUNK,), jnp.int32),
            pltpu.VMEM((_CHUNK,), jnp.int32),
            pltpu.SemaphoreType.DMA,
        ],
    )
    def combine(os_hbm, pos_hbm, out_hbm, r0, r1, ob, idx0, idx1, sem):
        wid = lax.axis_index("s") * 2 + lax.axis_index("c")
        rws = (_NW * tok_pw) // 128
        for c in range(tok_pw // _CHUNK):
            tb = wid * tok_pw + c * _CHUNK
            lk = c * _CHUNK
            g0 = pltpu.async_copy(pos_hbm.at[wid, pl.ds(lk, _CHUNK)], idx0,
                                  sem)
            g1 = pltpu.async_copy(pos_hbm.at[rws + wid, pl.ds(lk, _CHUNK)],
                                  idx1, sem)
            g0.wait()
            g1.wait()
            h0 = pltpu.async_copy(os_hbm.at[idx0], r0, sem)
            h1 = pltpu.async_copy(os_hbm.at[idx1], r1, sem)
            h0.wait()
            h1.wait()
            for i in range(_CHUNK):
                def vbody(v, _):
                    sl = pl.ds(v * 16, 16)
                    ob[i, sl] = r0[i, sl] + r1[i, sl]
                    return 0
                lax.fori_loop(0, dm // 16, vbody, 0)
            pltpu.sync_copy(ob, out_hbm.at[pl.ds(tb, _CHUNK)])

    return combine


def kernel(x, gate_w, w1, b1, w2, b2):
    bsz, seq, dm = x.shape
    n_exp, ff, _ = w1.shape
    n_tok = bsz * seq
    n_asn = n_tok * 2
    x2 = x.reshape(n_tok, dm)
    t = _TILE
    n_tiles = n_asn // t + n_exp
    mp = n_tiles * t

    rws = n_tok // 128
    router = pl.pallas_call(
        _router_body,
        out_shape=(
            jax.ShapeDtypeStruct((2 * rws, 128), jnp.float32),
            jax.ShapeDtypeStruct((2 * rws, 128), jnp.int32),
            jax.ShapeDtypeStruct((2, n_exp), jnp.int32),
            jax.ShapeDtypeStruct((1, 1), jnp.float32),
        ),
    )
    wq, pos, tr, aux = router(x2, gate_w)

    scatter = _make_scatter(n_tok, dm, mp, rws)
    xs, ws = scatter(x2, pos, wq)
    xsb = xs.astype(jnp.bfloat16)

    ft_n = ff // _FFB
    gmm = pl.pallas_call(
        _gmm_body,
        grid_spec=pltpu.PrefetchScalarGridSpec(
            num_scalar_prefetch=1,
            grid=(n_exp, ft_n),
            in_specs=[
                pl.BlockSpec((mp, dm), lambda e, ft, tr: (0, 0)),
                pl.BlockSpec((1, _FFB, dm), lambda e, ft, tr: (e, ft, 0)),
                pl.BlockSpec((1, 1, 1, _FFB), lambda e, ft, tr: (e, ft, 0, 0)),
                pl.BlockSpec((1, dm, _FFB), lambda e, ft, tr: (e, 0, ft)),
                pl.BlockSpec((1, 1, dm), lambda e, ft, tr: (e, 0, 0)),
                pl.BlockSpec((mp, 1), lambda e, ft, tr: (0, 0)),
            ],
            out_specs=pl.BlockSpec((mp, dm), lambda e, ft, tr: (0, 0)),
        ),
        out_shape=jax.ShapeDtypeStruct((mp, dm), jnp.float32),
        compiler_params=pltpu.CompilerParams(
            dimension_semantics=("arbitrary", "arbitrary"),
            vmem_limit_bytes=62 * 1024 * 1024),
    )
    os = gmm(tr, xsb, w1, b1.reshape(n_exp, ft_n, 1, _FFB),
             w2, b2.reshape(n_exp, 1, dm), ws.reshape(mp, 1))

    combine = _make_combine(n_tok, dm, mp)
    out = combine(os, pos)
    return out.reshape(bsz, seq, dm), aux[0, 0]


# speedup vs baseline: 4.2505x; 4.2505x over previous
import functools

import jax
import jax.numpy as jnp
from jax import lax
from jax.experimental import pallas as pl
from jax.experimental.pallas import tpu as pltpu
from jax.experimental.pallas import tpu_sc as plsc

_TILE = 512
_FFB = 1024
_NW = 32
_CHUNK = 32


def _router_body(x_ref, gw_ref, wq_ref, pos_ref, te_ref, aux_ref):
    x = x_ref[...]
    gw = gw_ref[...]
    n_tok = x.shape[0]
    n_exp = gw.shape[0]
    logits = jax.lax.dot_general(x, gw, (((1,), (1,)), ((), ())),
                                 preferred_element_type=jnp.float32)
    mx = jnp.max(logits, axis=-1, keepdims=True)
    ex = jnp.exp(logits - mx)
    probs = ex / jnp.sum(ex, axis=-1, keepdims=True)
    iot = jax.lax.broadcasted_iota(jnp.int32, probs.shape, 1)
    m1 = jnp.max(probs, axis=-1, keepdims=True)
    a1 = jnp.min(jnp.where(probs == m1, iot, n_exp), axis=-1, keepdims=True)
    sel1 = iot == a1
    pm = jnp.where(sel1, -1.0, probs)
    m2 = jnp.max(pm, axis=-1, keepdims=True)
    a2 = jnp.min(jnp.where(pm == m2, iot, n_exp), axis=-1, keepdims=True)
    sel2 = iot == a2
    wsum = m1 + m2
    cnt = jnp.sum(sel1.astype(jnp.float32) + sel2.astype(jnp.float32),
                  axis=0, keepdims=True)
    pmean = jnp.mean(probs, axis=0, keepdims=True)
    f = cnt / (n_tok * 2.0)
    aux_ref[...] = jnp.sum(f * pmean, keepdims=True).reshape(1, 1) * n_exp

    lanes = 128
    rws = n_tok // lanes
    ea = jnp.concatenate([jnp.reshape(a1, (rws, lanes)),
                          jnp.reshape(a2, (rws, lanes))], axis=0)
    wq_ref[...] = jnp.concatenate(
        [jnp.reshape(m1 / wsum, (rws, lanes)),
         jnp.reshape(m2 / wsum, (rws, lanes))], axis=0)
    rows = 2 * rws
    n_tiles = te_ref.shape[1]

    ci = cnt.astype(jnp.int32)
    pc = ((ci + _TILE - 1) // _TILE) * _TILE
    run = jnp.zeros((1, 1), jnp.int32)
    poffs, pcums = [], []
    for e in range(n_exp):
        poffs.append(run)
        run = run + pc[:, e:e + 1]
        pcums.append(run)

    li = lax.broadcasted_iota(jnp.int32, (lanes, lanes), 0)
    lj = lax.broadcasted_iota(jnp.int32, (lanes, lanes), 1)
    lt_inc = (li <= lj).astype(jnp.float32)
    ri = lax.broadcasted_iota(jnp.int32, (rows, rows), 0)
    rj = lax.broadcasted_iota(jnp.int32, (rows, rows), 1)
    rt_exc = (ri > rj).astype(jnp.float32)

    pos = jnp.zeros(ea.shape, jnp.int32)
    for e in range(n_exp):
        m = (ea == e).astype(jnp.float32)
        lane_c = jnp.dot(m, lt_inc,
                         preferred_element_type=jnp.float32)
        row_tot = lane_c[:, lanes - 1:lanes]
        row_off = jnp.dot(rt_exc, row_tot,
                          preferred_element_type=jnp.float32)
        rank = (row_off + lane_c - 1.0).astype(jnp.int32)
        pos = jnp.where(ea == e, poffs[e] + rank, pos)
    pos_ref[...] = pos

    ti = lax.broadcasted_iota(jnp.int32, (1, n_tiles), 1) * _TILE
    te = jnp.zeros((1, n_tiles), jnp.int32)
    for e in range(n_exp):
        te = te + (ti >= pcums[e]).astype(jnp.int32)
    te_ref[...] = jnp.minimum(te, n_exp - 1)


def _gmm_body(te_ref, xs_ref, w1_ref, b1_ref, w2_ref, b2_ref, ws_ref, os_ref):
    ft = pl.program_id(1)
    xs = xs_ref[...]
    h = jax.lax.dot_general(xs, w1_ref[0], (((1,), (1,)), ((), ())),
                            preferred_element_type=jnp.float32)
    h = jnp.maximum(h + b1_ref[0, 0], 0.0)
    contrib = jax.lax.dot_general(h, w2_ref[0], (((1,), (1,)), ((), ())),
                                  preferred_element_type=jnp.float32)
    ws = ws_ref[...]

    @pl.when(ft == 0)
    def _init():
        os_ref[...] = (contrib + b2_ref[0]) * ws

    @pl.when(ft != 0)
    def _acc():
        os_ref[...] += contrib * ws


def _make_scatter(n_tok, dm, mp, rws):
    mesh = plsc.VectorSubcoreMesh(core_axis_name="c", subcore_axis_name="s")
    tok_pw = n_tok // _NW
    ck = 64

    @functools.partial(
        pl.kernel, mesh=mesh,
        out_type=(jax.ShapeDtypeStruct((mp, dm), jnp.float32),
                  jax.ShapeDtypeStruct((mp,), jnp.float32)),
        scratch_types=[
            pltpu.VMEM((ck, dm), jnp.float32),
            pltpu.VMEM((ck,), jnp.int32),
            pltpu.VMEM((ck,), jnp.int32),
            pltpu.VMEM((ck,), jnp.float32),
            pltpu.VMEM((ck,), jnp.float32),
            pltpu.SemaphoreType.DMA,
            pltpu.SemaphoreType.DMA,
        ],
    )
    def scatter(x_hbm, pos_hbm, wq_hbm, xs_hbm, ws_hbm,
                xbuf, idx0, idx1, wb0, wb1, lsem, ssem):
        wid = lax.axis_index("s") * 2 + lax.axis_index("c")
        for c in range(tok_pw // ck):
            tb = wid * tok_pw + c * ck
            lk = c * ck
            pltpu.sync_copy(x_hbm.at[pl.ds(tb, ck)], xbuf)
            pltpu.sync_copy(pos_hbm.at[wid, pl.ds(lk, ck)], idx0)
            pltpu.sync_copy(pos_hbm.at[rws + wid, pl.ds(lk, ck)], idx1)
            pltpu.sync_copy(wq_hbm.at[wid, pl.ds(lk, ck)], wb0)
            pltpu.sync_copy(wq_hbm.at[rws + wid, pl.ds(lk, ck)], wb1)
            s1 = pltpu.async_copy(xbuf, xs_hbm.at[idx0], ssem)
            s2 = pltpu.async_copy(xbuf, xs_hbm.at[idx1], ssem)
            s3 = pltpu.async_copy(wb0, ws_hbm.at[idx0], ssem)
            s4 = pltpu.async_copy(wb1, ws_hbm.at[idx1], ssem)
            for h in (s1, s2, s3, s4):
                h.wait()

    return scatter


def _make_combine(n_tok, dm, mp):
    mesh = plsc.VectorSubcoreMesh(core_axis_name="c", subcore_axis_name="s")
    tok_pw = n_tok // _NW

    @functools.partial(
        pl.kernel, mesh=mesh,
        out_type=jax.ShapeDtypeStruct((n_tok, dm), jnp.float32),
        scratch_types=[
            pltpu.VMEM((_CHUNK, dm), jnp.float32),
            pltpu.VMEM((_CHUNK, dm), jnp.float32),
            pltpu.VMEM((_CHUNK, dm), jnp.float32),
            pltpu.VMEM((_CHUNK,), jnp.int32),
            pltpu.VMEM((_CHUNK,), jnp.int32),
            pltpu.SemaphoreType.DMA,
        ],
    )
    def combine(os_hbm, pos_hbm, out_hbm, r0, r1, ob, idx0, idx1, sem):
        wid = lax.axis_index("s") * 2 + lax.axis_index("c")
        rws = (_NW * tok_pw) // 128
        for c in range(tok_pw // _CHUNK):
            tb = wid * tok_pw + c * _CHUNK
            lk = c * _CHUNK
            g0 = pltpu.async_copy(pos_hbm.at[wid, pl.ds(lk, _CHUNK)], idx0,
                                  sem)
            g1 = pltpu.async_copy(pos_hbm.at[rws + wid, pl.ds(lk, _CHUNK)],
                                  idx1, sem)
            g0.wait()
            g1.wait()
            h0 = pltpu.async_copy(os_hbm.at[idx0], r0, sem)
            h1 = pltpu.async_copy(os_hbm.at[idx1], r1, sem)
            h0.wait()
            h1.wait()
            for i in range(_CHUNK):
                def vbody(v, _):
                    sl = pl.ds(v * 16, 16)
                    ob[i, sl] = r0[i, sl] + r1[i, sl]
                    return 0
                lax.fori_loop(0, dm // 16, vbody, 0)
            pltpu.sync_copy(ob, out_hbm.at[pl.ds(tb, _CHUNK)])

    return combine


def kernel(x, gate_w, w1, b1, w2, b2):
    bsz, seq, dm = x.shape
    n_exp, ff, _ = w1.shape
    n_tok = bsz * seq
    n_asn = n_tok * 2
    x2 = x.reshape(n_tok, dm)
    t = _TILE
    n_tiles = n_asn // t + n_exp
    mp = n_tiles * t

    rws = n_tok // 128
    router = pl.pallas_call(
        _router_body,
        out_shape=(
            jax.ShapeDtypeStruct((2 * rws, 128), jnp.float32),
            jax.ShapeDtypeStruct((2 * rws, 128), jnp.int32),
            jax.ShapeDtypeStruct((1, n_tiles), jnp.int32),
            jax.ShapeDtypeStruct((1, 1), jnp.float32),
        ),
    )
    wq, pos, te, aux = router(x2, gate_w)

    scatter = _make_scatter(n_tok, dm, mp, rws)
    xs, ws = scatter(x2, pos, wq)

    ft_n = ff // _FFB
    gmm = pl.pallas_call(
        _gmm_body,
        grid_spec=pltpu.PrefetchScalarGridSpec(
            num_scalar_prefetch=1,
            grid=(n_tiles, ft_n),
            in_specs=[
                pl.BlockSpec((t, dm), lambda mt, ft, te: (mt, 0)),
                pl.BlockSpec((1, _FFB, dm),
                             lambda mt, ft, te: (te[0, mt], ft, 0)),
                pl.BlockSpec((1, 1, 1, _FFB),
                             lambda mt, ft, te: (te[0, mt], ft, 0, 0)),
                pl.BlockSpec((1, dm, _FFB),
                             lambda mt, ft, te: (te[0, mt], 0, ft)),
                pl.BlockSpec((1, 1, dm),
                             lambda mt, ft, te: (te[0, mt], 0, 0)),
                pl.BlockSpec((t, 1), lambda mt, ft, te: (mt, 0)),
            ],
            out_specs=pl.BlockSpec((t, dm), lambda mt, ft, te: (mt, 0)),
        ),
        out_shape=jax.ShapeDtypeStruct((mp, dm), jnp.float32),
        compiler_params=pltpu.CompilerParams(
            dimension_semantics=("arbitrary", "arbitrary")),
    )
    os = gmm(te, xs, w1, b1.reshape(n_exp, ft_n, 1, _FFB),
             w2, b2.reshape(n_exp, 1, dm), ws.reshape(mp, 1))

    combine = _make_combine(n_tok, dm, mp)
    out = combine(os, pos)
    return out.reshape(bsz, seq, dm), aux[0, 0]


# speedup vs baseline: 4.5328x; 1.0664x over previous
import functools

import jax
import jax.numpy as jnp
from jax import lax
from jax.experimental import pallas as pl
from jax.experimental.pallas import tpu as pltpu
from jax.experimental.pallas import tpu_sc as plsc

_TILE = 512
_FFB = 2048
_NW = 32
_CHUNK = 32


def _router_body(x_ref, gw_ref, wq_ref, pos_ref, te_ref, aux_ref):
    x = x_ref[...]
    gw = gw_ref[...]
    n_tok = x.shape[0]
    n_exp = gw.shape[0]
    logits = jax.lax.dot_general(x, gw, (((1,), (1,)), ((), ())),
                                 preferred_element_type=jnp.float32)
    mx = jnp.max(logits, axis=-1, keepdims=True)
    ex = jnp.exp(logits - mx)
    probs = ex / jnp.sum(ex, axis=-1, keepdims=True)
    iot = jax.lax.broadcasted_iota(jnp.int32, probs.shape, 1)
    m1 = jnp.max(probs, axis=-1, keepdims=True)
    a1 = jnp.min(jnp.where(probs == m1, iot, n_exp), axis=-1, keepdims=True)
    sel1 = iot == a1
    pm = jnp.where(sel1, -1.0, probs)
    m2 = jnp.max(pm, axis=-1, keepdims=True)
    a2 = jnp.min(jnp.where(pm == m2, iot, n_exp), axis=-1, keepdims=True)
    sel2 = iot == a2
    wsum = m1 + m2
    cnt = jnp.sum(sel1.astype(jnp.float32) + sel2.astype(jnp.float32),
                  axis=0, keepdims=True)
    pmean = jnp.mean(probs, axis=0, keepdims=True)
    f = cnt / (n_tok * 2.0)
    aux_ref[...] = jnp.sum(f * pmean, keepdims=True).reshape(1, 1) * n_exp

    lanes = 128
    rws = n_tok // lanes
    ea = jnp.concatenate([jnp.reshape(a1, (rws, lanes)),
                          jnp.reshape(a2, (rws, lanes))], axis=0)
    wq_ref[...] = jnp.concatenate(
        [jnp.reshape(m1 / wsum, (rws, lanes)),
         jnp.reshape(m2 / wsum, (rws, lanes))], axis=0)
    rows = 2 * rws
    n_tiles = te_ref.shape[1]

    ci = cnt.astype(jnp.int32)
    pc = ((ci + _TILE - 1) // _TILE) * _TILE
    run = jnp.zeros((1, 1), jnp.int32)
    poffs, pcums = [], []
    for e in range(n_exp):
        poffs.append(run)
        run = run + pc[:, e:e + 1]
        pcums.append(run)

    li = lax.broadcasted_iota(jnp.int32, (lanes, lanes), 0)
    lj = lax.broadcasted_iota(jnp.int32, (lanes, lanes), 1)
    lt_inc = (li <= lj).astype(jnp.float32)
    ri = lax.broadcasted_iota(jnp.int32, (rows, rows), 0)
    rj = lax.broadcasted_iota(jnp.int32, (rows, rows), 1)
    rt_exc = (ri > rj).astype(jnp.float32)

    pos = jnp.zeros(ea.shape, jnp.int32)
    for e in range(n_exp):
        m = (ea == e).astype(jnp.float32)
        lane_c = jnp.dot(m, lt_inc,
                         preferred_element_type=jnp.float32)
        row_tot = lane_c[:, lanes - 1:lanes]
        row_off = jnp.dot(rt_exc, row_tot,
                          preferred_element_type=jnp.float32)
        rank = (row_off + lane_c - 1.0).astype(jnp.int32)
        pos = jnp.where(ea == e, poffs[e] + rank, pos)
    pos_ref[...] = pos

    ti = lax.broadcasted_iota(jnp.int32, (1, n_tiles), 1) * _TILE
    te = jnp.zeros((1, n_tiles), jnp.int32)
    for e in range(n_exp):
        te = te + (ti >= pcums[e]).astype(jnp.int32)
    te_ref[...] = jnp.minimum(te, n_exp - 1)


def _gmm_body(te_ref, xs_ref, w1_ref, b1_ref, w2_ref, b2_ref, ws_ref, os_ref):
    ft = pl.program_id(1)
    xs = xs_ref[...]
    h = jax.lax.dot_general(xs, w1_ref[0], (((1,), (1,)), ((), ())),
                            preferred_element_type=jnp.float32)
    h = jnp.maximum(h + b1_ref[0, 0], 0.0)
    contrib = jax.lax.dot_general(h, w2_ref[0], (((1,), (1,)), ((), ())),
                                  preferred_element_type=jnp.float32)
    ws = ws_ref[...]

    @pl.when(ft == 0)
    def _init():
        os_ref[...] = (contrib + b2_ref[0]) * ws

    @pl.when(ft != 0)
    def _acc():
        os_ref[...] += contrib * ws


def _make_scatter(n_tok, dm, mp, rws):
    mesh = plsc.VectorSubcoreMesh(core_axis_name="c", subcore_axis_name="s")
    tok_pw = n_tok // _NW
    ck = 64

    @functools.partial(
        pl.kernel, mesh=mesh,
        out_type=(jax.ShapeDtypeStruct((mp, dm), jnp.float32),
                  jax.ShapeDtypeStruct((mp,), jnp.float32)),
        scratch_types=[
            pltpu.VMEM((ck, dm), jnp.float32),
            pltpu.VMEM((ck,), jnp.int32),
            pltpu.VMEM((ck,), jnp.int32),
            pltpu.VMEM((ck,), jnp.float32),
            pltpu.VMEM((ck,), jnp.float32),
            pltpu.SemaphoreType.DMA,
            pltpu.SemaphoreType.DMA,
        ],
    )
    def scatter(x_hbm, pos_hbm, wq_hbm, xs_hbm, ws_hbm,
                xbuf, idx0, idx1, wb0, wb1, lsem, ssem):
        wid = lax.axis_index("s") * 2 + lax.axis_index("c")
        for c in range(tok_pw // ck):
            tb = wid * tok_pw + c * ck
            lk = c * ck
            pltpu.sync_copy(x_hbm.at[pl.ds(tb, ck)], xbuf)
            pltpu.sync_copy(pos_hbm.at[wid, pl.ds(lk, ck)], idx0)
            pltpu.sync_copy(pos_hbm.at[rws + wid, pl.ds(lk, ck)], idx1)
            pltpu.sync_copy(wq_hbm.at[wid, pl.ds(lk, ck)], wb0)
            pltpu.sync_copy(wq_hbm.at[rws + wid, pl.ds(lk, ck)], wb1)
            s1 = pltpu.async_copy(xbuf, xs_hbm.at[idx0], ssem)
            s2 = pltpu.async_copy(xbuf, xs_hbm.at[idx1], ssem)
            s3 = pltpu.async_copy(wb0, ws_hbm.at[idx0], ssem)
            s4 = pltpu.async_copy(wb1, ws_hbm.at[idx1], ssem)
            for h in (s1, s2, s3, s4):
                h.wait()

    return scatter


def _make_combine(n_tok, dm, mp):
    mesh = plsc.VectorSubcoreMesh(core_axis_name="c", subcore_axis_name="s")
    tok_pw = n_tok // _NW

    @functools.partial(
        pl.kernel, mesh=mesh,
        out_type=jax.ShapeDtypeStruct((n_tok, dm), jnp.float32),
        scratch_types=[
            pltpu.VMEM((_CHUNK, dm), jnp.float32),
            pltpu.VMEM((_CHUNK, dm), jnp.float32),
            pltpu.VMEM((_CHUNK, dm), jnp.float32),
            pltpu.VMEM((_CHUNK,), jnp.int32),
            pltpu.VMEM((_CHUNK,), jnp.int32),
            pltpu.SemaphoreType.DMA,
        ],
    )
    def combine(os_hbm, pos_hbm, out_hbm, r0, r1, ob, idx0, idx1, sem):
        wid = lax.axis_index("s") * 2 + lax.axis_index("c")
        rws = (_NW * tok_pw) // 128
        for c in range(tok_pw // _CHUNK):
            tb = wid * tok_pw + c * _CHUNK
            lk = c * _CHUNK
            g0 = pltpu.async_copy(pos_hbm.at[wid, pl.ds(lk, _CHUNK)], idx0,
                                  sem)
            g1 = pltpu.async_copy(pos_hbm.at[rws + wid, pl.ds(lk, _CHUNK)],
                                  idx1, sem)
            g0.wait()
            g1.wait()
            h0 = pltpu.async_copy(os_hbm.at[idx0], r0, sem)
            h1 = pltpu.async_copy(os_hbm.at[idx1], r1, sem)
            h0.wait()
            h1.wait()
            for i in range(_CHUNK):
                def vbody(v, _):
                    sl = pl.ds(v * 16, 16)
                    ob[i, sl] = r0[i, sl] + r1[i, sl]
                    return 0
                lax.fori_loop(0, dm // 16, vbody, 0)
            pltpu.sync_copy(ob, out_hbm.at[pl.ds(tb, _CHUNK)])

    return combine


def kernel(x, gate_w, w1, b1, w2, b2):
    bsz, seq, dm = x.shape
    n_exp, ff, _ = w1.shape
    n_tok = bsz * seq
    n_asn = n_tok * 2
    x2 = x.reshape(n_tok, dm)
    t = _TILE
    n_tiles = n_asn // t + n_exp
    mp = n_tiles * t

    rws = n_tok // 128
    router = pl.pallas_call(
        _router_body,
        out_shape=(
            jax.ShapeDtypeStruct((2 * rws, 128), jnp.float32),
            jax.ShapeDtypeStruct((2 * rws, 128), jnp.int32),
            jax.ShapeDtypeStruct((1, n_tiles), jnp.int32),
            jax.ShapeDtypeStruct((1, 1), jnp.float32),
        ),
    )
    wq, pos, te, aux = router(x2, gate_w)

    scatter = _make_scatter(n_tok, dm, mp, rws)
    xs, ws = scatter(x2, pos, wq)

    ft_n = ff // _FFB
    gmm = pl.pallas_call(
        _gmm_body,
        grid_spec=pltpu.PrefetchScalarGridSpec(
            num_scalar_prefetch=1,
            grid=(n_tiles, ft_n),
            in_specs=[
                pl.BlockSpec((t, dm), lambda mt, ft, te: (mt, 0)),
                pl.BlockSpec((1, _FFB, dm),
                             lambda mt, ft, te: (te[0, mt], ft, 0)),
                pl.BlockSpec((1, 1, 1, _FFB),
                             lambda mt, ft, te: (te[0, mt], ft, 0, 0)),
                pl.BlockSpec((1, dm, _FFB),
                             lambda mt, ft, te: (te[0, mt], 0, ft)),
                pl.BlockSpec((1, 1, dm),
                             lambda mt, ft, te: (te[0, mt], 0, 0)),
                pl.BlockSpec((t, 1), lambda mt, ft, te: (mt, 0)),
            ],
            out_specs=pl.BlockSpec((t, dm), lambda mt, ft, te: (mt, 0)),
        ),
        out_shape=jax.ShapeDtypeStruct((mp, dm), jnp.float32),
        compiler_params=pltpu.CompilerParams(
            dimension_semantics=("arbitrary", "arbitrary")),
    )
    os = gmm(te, xs, w1, b1.reshape(n_exp, ft_n, 1, _FFB),
             w2, b2.reshape(n_exp, 1, dm), ws.reshape(mp, 1))

    combine = _make_combine(n_tok, dm, mp)
    out = combine(os, pos)
    return out.reshape(bsz, seq, dm), aux[0, 0]
